# baseline (device time: 236481 ns/iter reference)
import jax
import jax.numpy as jnp
from jax import lax
from jax.experimental import pallas as pl
from jax.experimental.pallas import tpu as pltpu

N_DEV = 16


def kernel(x, w_mat):
    m, k_shard = x.shape
    _, n = w_mat.shape
    ch = m // N_DEV

    def body(x_ref, w_ref, out_ref, p_ref, stage_ref, rs_recv,
             rs_send_sems, rs_recv_sems, ag_send_sems, ag_recv_sems):
        d = lax.axis_index("i")
        left = lax.rem(d + (N_DEV - 1), N_DEV)
        right = lax.rem(d + 1, N_DEV)

        barrier_sem = pltpu.get_barrier_semaphore()
        for nbr in (left, right):
            pl.semaphore_signal(
                barrier_sem, inc=1,
                device_id=(nbr,), device_id_type=pl.DeviceIdType.MESH,
            )
        pl.semaphore_wait(barrier_sem, 2)

        p_ref[...] = jnp.dot(
            x_ref[...], w_ref[...], preferred_element_type=jnp.float32
        )

        for t in range(N_DEV - 1):
            c = lax.rem(d + (N_DEV - t), N_DEV)
            val = p_ref[pl.ds(c * ch, ch), :]
            if t > 0:
                val = val + rs_recv[t - 1].astype(jnp.float32)
            stage_ref[t % 2] = val.astype(jnp.bfloat16)
            rdma = pltpu.make_async_remote_copy(
                src_ref=stage_ref.at[t % 2],
                dst_ref=rs_recv.at[t],
                send_sem=rs_send_sems.at[t],
                recv_sem=rs_recv_sems.at[t],
                device_id=(right,),
                device_id_type=pl.DeviceIdType.MESH,
            )
            rdma.start()
            rdma.wait()

        c_fin = lax.rem(d + 1, N_DEV)
        fin = (p_ref[pl.ds(c_fin * ch, ch), :]
               + rs_recv[N_DEV - 2].astype(jnp.float32))
        out_ref[pl.ds(c_fin * ch, ch), :] = jnp.maximum(fin, 0.0).astype(
            jnp.bfloat16
        )

        for s in range(N_DEV - 1):
            c = lax.rem(d + (N_DEV + 1 - s), N_DEV)
            rows = pl.ds(c * ch, ch)
            rdma = pltpu.make_async_remote_copy(
                src_ref=out_ref.at[rows, :],
                dst_ref=out_ref.at[rows, :],
                send_sem=ag_send_sems.at[s],
                recv_sem=ag_recv_sems.at[s],
                device_id=(right,),
                device_id_type=pl.DeviceIdType.MESH,
            )
            rdma.start()
            rdma.wait()

    return pl.pallas_call(
        body,
        out_shape=jax.ShapeDtypeStruct((m, n), jnp.bfloat16),
        in_specs=[
            pl.BlockSpec(memory_space=pltpu.VMEM),
            pl.BlockSpec(memory_space=pltpu.VMEM),
        ],
        out_specs=pl.BlockSpec(memory_space=pltpu.VMEM),
        scratch_shapes=[
            pltpu.VMEM((m, n), jnp.float32),
            pltpu.VMEM((2, ch, n), jnp.bfloat16),
            pltpu.VMEM((N_DEV - 1, ch, n), jnp.bfloat16),
            pltpu.SemaphoreType.DMA((N_DEV - 1,)),
            pltpu.SemaphoreType.DMA((N_DEV - 1,)),
            pltpu.SemaphoreType.DMA((N_DEV - 1,)),
            pltpu.SemaphoreType.DMA((N_DEV - 1,)),
        ],
        compiler_params=pltpu.CompilerParams(collective_id=0),
    )(x, w_mat)


# device time: 231633 ns/iter; 1.0209x vs baseline; 1.0209x over previous
import jax
import jax.numpy as jnp
from jax import lax
from jax.experimental import pallas as pl
from jax.experimental.pallas import tpu as pltpu

N_DEV = 16


def kernel(x, w_mat):
    m, k_shard = x.shape
    _, n = w_mat.shape
    ch = m // N_DEV

    def body(x_ref, w_ref, out_ref, pb_ref, rs_recv,
             rs_send_sems, rs_recv_sems, ag_send_sems, ag_recv_sems):
        d = lax.axis_index("i")

        def rows_of(c):
            return pl.ds(c * ch, ch)

        barrier_sem = pltpu.get_barrier_semaphore()
        for r in range(1, N_DEV):
            peer = lax.rem(d + r, N_DEV)
            pl.semaphore_signal(
                barrier_sem, inc=1,
                device_id=(peer,), device_id_type=pl.DeviceIdType.MESH,
            )
        pl.semaphore_wait(barrier_sem, N_DEV - 1)

        pb_ref[...] = jnp.dot(
            x_ref[...], w_ref[...], preferred_element_type=jnp.float32
        ).astype(jnp.bfloat16)

        for r in range(1, N_DEV):
            c = lax.rem(d + (N_DEV - r), N_DEV)
            rdma = pltpu.make_async_remote_copy(
                src_ref=pb_ref.at[rows_of(c), :],
                dst_ref=rs_recv.at[r],
                send_sem=rs_send_sems.at[r],
                recv_sem=rs_recv_sems.at[r],
                device_id=(c,),
                device_id_type=pl.DeviceIdType.MESH,
            )
            rdma.start()

        acc = pb_ref[rows_of(d), :].astype(jnp.float32)
        for r in range(1, N_DEV):
            recv = pltpu.make_async_remote_copy(
                src_ref=pb_ref.at[rows_of(d), :],
                dst_ref=rs_recv.at[r],
                send_sem=rs_send_sems.at[r],
                recv_sem=rs_recv_sems.at[r],
                device_id=(d,),
                device_id_type=pl.DeviceIdType.MESH,
            )
            recv.wait_recv()
            acc = acc + rs_recv[r].astype(jnp.float32)
        out_ref[rows_of(d), :] = jnp.maximum(acc, 0.0).astype(jnp.bfloat16)

        for r in range(1, N_DEV):
            t = lax.rem(d + (N_DEV - r), N_DEV)
            rdma = pltpu.make_async_remote_copy(
                src_ref=out_ref.at[rows_of(d), :],
                dst_ref=out_ref.at[rows_of(d), :],
                send_sem=ag_send_sems.at[r],
                recv_sem=ag_recv_sems.at[r],
                device_id=(t,),
                device_id_type=pl.DeviceIdType.MESH,
            )
            rdma.start()

        for r in range(1, N_DEV):
            snd = pltpu.make_async_remote_copy(
                src_ref=pb_ref.at[rows_of(d), :],
                dst_ref=rs_recv.at[r],
                send_sem=rs_send_sems.at[r],
                recv_sem=rs_recv_sems.at[r],
                device_id=(d,),
                device_id_type=pl.DeviceIdType.MESH,
            )
            snd.wait_send()

        for r in range(1, N_DEV):
            o = lax.rem(d + r, N_DEV)
            recv = pltpu.make_async_remote_copy(
                src_ref=out_ref.at[rows_of(d), :],
                dst_ref=out_ref.at[rows_of(o), :],
                send_sem=ag_send_sems.at[r],
                recv_sem=ag_recv_sems.at[r],
                device_id=(o,),
                device_id_type=pl.DeviceIdType.MESH,
            )
            recv.wait_recv()

        for r in range(1, N_DEV):
            snd = pltpu.make_async_remote_copy(
                src_ref=out_ref.at[rows_of(d), :],
                dst_ref=out_ref.at[rows_of(d), :],
                send_sem=ag_send_sems.at[r],
                recv_sem=ag_recv_sems.at[r],
                device_id=(d,),
                device_id_type=pl.DeviceIdType.MESH,
            )
            snd.wait_send()

    return pl.pallas_call(
        body,
        out_shape=jax.ShapeDtypeStruct((m, n), jnp.bfloat16),
        in_specs=[
            pl.BlockSpec(memory_space=pltpu.VMEM),
            pl.BlockSpec(memory_space=pltpu.VMEM),
        ],
        out_specs=pl.BlockSpec(memory_space=pltpu.VMEM),
        scratch_shapes=[
            pltpu.VMEM((m, n), jnp.bfloat16),
            pltpu.VMEM((N_DEV, ch, n), jnp.bfloat16),
            pltpu.SemaphoreType.DMA((N_DEV,)),
            pltpu.SemaphoreType.DMA((N_DEV,)),
            pltpu.SemaphoreType.DMA((N_DEV,)),
            pltpu.SemaphoreType.DMA((N_DEV,)),
        ],
        compiler_params=pltpu.CompilerParams(collective_id=0),
    )(x, w_mat)


# device time: 107293 ns/iter; 2.2041x vs baseline; 2.1589x over previous
import jax
import jax.numpy as jnp
from jax import lax
from jax.experimental import pallas as pl
from jax.experimental.pallas import tpu as pltpu

N_DEV = 16
M = 2048

MASK_X, MASK_Y, MASK_Z1, MASK_Z2 = 1, 3, 4, 8

STREAMS = (
    (0, 768, (MASK_X, MASK_Y, MASK_Z1, MASK_Z2)),
    (768, 640, (MASK_Y, MASK_Z1, MASK_Z2, MASK_X)),
    (1408, 640, (MASK_Z1, MASK_Z2, MASK_X, MASK_Y)),
)

LENS = (1024, 512, 256, 128)
RECV_OFF = (0, 1024, 1536, 1792)
STAGE_OFF = (0, 0, 512, 768)


def _parity(v):
    return (v ^ (v >> 1) ^ (v >> 2) ^ (v >> 3)) & 1


def _functionals(masks):
    def par(x):
        return bin(x).count("1") & 1

    fs = []
    for j in range(4):
        for phi in range(16):
            if par(phi & masks[j]) == 1 and all(
                par(phi & masks[k]) == 0 for k in range(j + 1, 4)
            ):
                fs.append(phi)
                break
    assert len(fs) == j + 1
    return fs


def kernel(x, w_mat):
    m, k_shard = x.shape
    _, n = w_mat.shape

    def body(x_ref, w_ref, out_ref, pb_ref, acc_ref, stage_ref, recv_ref,
             rs_send_sems, rs_recv_sems, ag_send_sems, ag_recv_sems):
        d = lax.axis_index("i")

        barrier_sem = pltpu.get_barrier_semaphore()
        for mk in (MASK_X, MASK_Y, MASK_Z1, MASK_Z2):
            pl.semaphore_signal(
                barrier_sem, inc=1,
                device_id=(d ^ mk,), device_id_type=pl.DeviceIdType.MESH,
            )
        pl.semaphore_wait(barrier_sem, 4)

        pb_ref[...] = jnp.dot(
            x_ref[...], w_ref[...], preferred_element_type=jnp.float32
        ).astype(jnp.bfloat16)

        sinfo = []
        for si, (c0, cw, masks) in enumerate(STREAMS):
            phis = _functionals(masks)
            h = [_parity(d & phi) for phi in phis]
            kstart = h[0] * 1024
            o = [0, h[1] * 512, h[1] * 512 + h[2] * 256]
            o.append(o[2] + h[3] * 128)
            sinfo.append(dict(c0=c0, cw=cw, masks=masks, h=h,
                              kstart=kstart, o=o))

        for j in range(4):
            for si, st in enumerate(sinfo):
                c0, cw, h = st["c0"], st["cw"], st["h"]
                partner = d ^ st["masks"][j]
                ln = LENS[j]
                if j == 0:
                    src = pb_ref.at[pl.ds((1 - h[0]) * 1024, 1024),
                                    pl.ds(c0, cw)]
                else:
                    prev = pltpu.make_async_remote_copy(
                        src_ref=pb_ref.at[pl.ds(0, LENS[j - 1]), pl.ds(c0, cw)],
                        dst_ref=recv_ref.at[pl.ds(RECV_OFF[j - 1], LENS[j - 1]),
                                            pl.ds(c0, cw)],
                        send_sem=rs_send_sems.at[si, j - 1],
                        recv_sem=rs_recv_sems.at[si, j - 1],
                        device_id=(d,),
                        device_id_type=pl.DeviceIdType.MESH,
                    )
                    prev.wait_recv()
                    rcv = recv_ref[pl.ds(RECV_OFF[j - 1], LENS[j - 1]),
                                   pl.ds(c0, cw)].astype(jnp.float32)
                    if j == 1:
                        base = pb_ref[pl.ds(st["kstart"] + 0, 1024),
                                      pl.ds(c0, cw)].astype(jnp.float32)
                        acc_ref[pl.ds(0, 1024), pl.ds(c0, cw)] = base + rcv
                    else:
                        cur = acc_ref[pl.ds(st["o"][j - 1], LENS[j - 1]),
                                      pl.ds(c0, cw)]
                        acc_ref[pl.ds(st["o"][j - 1], LENS[j - 1]),
                                pl.ds(c0, cw)] = cur + rcv
                    send_rel = st["o"][j - 1] + (1 - h[j]) * ln
                    stage_ref[pl.ds(STAGE_OFF[j], ln), pl.ds(c0, cw)] = (
                        acc_ref[pl.ds(send_rel, ln), pl.ds(c0, cw)]
                        .astype(jnp.bfloat16)
                    )
                    src = stage_ref.at[pl.ds(STAGE_OFF[j], ln), pl.ds(c0, cw)]
                rdma = pltpu.make_async_remote_copy(
                    src_ref=src,
                    dst_ref=recv_ref.at[pl.ds(RECV_OFF[j], ln), pl.ds(c0, cw)],
                    send_sem=rs_send_sems.at[si, j],
                    recv_sem=rs_recv_sems.at[si, j],
                    device_id=(partner,),
                    device_id_type=pl.DeviceIdType.MESH,
                )
                rdma.start()

        for si, st in enumerate(sinfo):
            c0, cw = st["c0"], st["cw"]
            last = pltpu.make_async_remote_copy(
                src_ref=pb_ref.at[pl.ds(0, 128), pl.ds(c0, cw)],
                dst_ref=recv_ref.at[pl.ds(RECV_OFF[3], 128), pl.ds(c0, cw)],
                send_sem=rs_send_sems.at[si, 3],
                recv_sem=rs_recv_sems.at[si, 3],
                device_id=(d,),
                device_id_type=pl.DeviceIdType.MESH,
            )
            last.wait_recv()
            fin = (acc_ref[pl.ds(st["o"][3], 128), pl.ds(c0, cw)]
                   + recv_ref[pl.ds(RECV_OFF[3], 128),
                              pl.ds(c0, cw)].astype(jnp.float32))
            g = st["kstart"] + st["o"][3]
            out_ref[pl.ds(g, 128), pl.ds(c0, cw)] = (
                jnp.maximum(fin, 0.0).astype(jnp.bfloat16)
            )

        for idx, j in enumerate((3, 2, 1, 0)):
            for si, st in enumerate(sinfo):
                c0, cw = st["c0"], st["cw"]
                partner = d ^ st["masks"][j]
                ln = LENS[j]
                blk_start = st["kstart"] + st["o"][j]
                if idx > 0:
                    pj = j + 1
                    prev = pltpu.make_async_remote_copy(
                        src_ref=out_ref.at[pl.ds(0, LENS[pj]), pl.ds(c0, cw)],
                        dst_ref=out_ref.at[
                            pl.ds(st["kstart"] + st["o"][pj], LENS[pj]),
                            pl.ds(c0, cw)],
                        send_sem=ag_send_sems.at[si, pj],
                        recv_sem=ag_recv_sems.at[si, pj],
                        device_id=(d,),
                        device_id_type=pl.DeviceIdType.MESH,
                    )
                    prev.wait_recv()
                rdma = pltpu.make_async_remote_copy(
                    src_ref=out_ref.at[pl.ds(blk_start, ln), pl.ds(c0, cw)],
                    dst_ref=out_ref.at[pl.ds(blk_start, ln), pl.ds(c0, cw)],
                    send_sem=ag_send_sems.at[si, j],
                    recv_sem=ag_recv_sems.at[si, j],
                    device_id=(partner,),
                    device_id_type=pl.DeviceIdType.MESH,
                )
                rdma.start()

        for si, st in enumerate(sinfo):
            c0, cw = st["c0"], st["cw"]
            fin = pltpu.make_async_remote_copy(
                src_ref=out_ref.at[pl.ds(0, 1024), pl.ds(c0, cw)],
                dst_ref=out_ref.at[pl.ds(st["kstart"], 1024), pl.ds(c0, cw)],
                send_sem=ag_send_sems.at[si, 0],
                recv_sem=ag_recv_sems.at[si, 0],
                device_id=(d,),
                device_id_type=pl.DeviceIdType.MESH,
            )
            fin.wait_recv()
        for si, st in enumerate(sinfo):
            c0, cw = st["c0"], st["cw"]
            for j in range(4):
                ln = LENS[j]
                rs_snd = pltpu.make_async_remote_copy(
                    src_ref=pb_ref.at[pl.ds(0, ln), pl.ds(c0, cw)],
                    dst_ref=recv_ref.at[pl.ds(RECV_OFF[j], ln), pl.ds(c0, cw)],
                    send_sem=rs_send_sems.at[si, j],
                    recv_sem=rs_recv_sems.at[si, j],
                    device_id=(d,),
                    device_id_type=pl.DeviceIdType.MESH,
                )
                rs_snd.wait_send()
                ag_snd = pltpu.make_async_remote_copy(
                    src_ref=out_ref.at[pl.ds(0, ln), pl.ds(c0, cw)],
                    dst_ref=out_ref.at[pl.ds(0, ln), pl.ds(c0, cw)],
                    send_sem=ag_send_sems.at[si, j],
                    recv_sem=ag_recv_sems.at[si, j],
                    device_id=(d,),
                    device_id_type=pl.DeviceIdType.MESH,
                )
                ag_snd.wait_send()

    n_streams = len(STREAMS)
    return pl.pallas_call(
        body,
        out_shape=jax.ShapeDtypeStruct((m, n), jnp.bfloat16),
        in_specs=[
            pl.BlockSpec(memory_space=pltpu.VMEM),
            pl.BlockSpec(memory_space=pltpu.VMEM),
        ],
        out_specs=pl.BlockSpec(memory_space=pltpu.VMEM),
        scratch_shapes=[
            pltpu.VMEM((m, n), jnp.bfloat16),
            pltpu.VMEM((1024, n), jnp.float32),
            pltpu.VMEM((1024, n), jnp.bfloat16),
            pltpu.VMEM((1920, n), jnp.bfloat16),
            pltpu.SemaphoreType.DMA((n_streams, 4)),
            pltpu.SemaphoreType.DMA((n_streams, 4)),
            pltpu.SemaphoreType.DMA((n_streams, 4)),
            pltpu.SemaphoreType.DMA((n_streams, 4)),
        ],
        compiler_params=pltpu.CompilerParams(collective_id=0),
    )(x, w_mat)


# device time: 103579 ns/iter; 2.2831x vs baseline; 1.0359x over previous
import jax
import jax.numpy as jnp
from jax import lax
from jax.experimental import pallas as pl
from jax.experimental.pallas import tpu as pltpu

N_DEV = 16
M = 2048

MASK_X, MASK_Y, MASK_Z1, MASK_Z2 = 1, 3, 4, 8

STREAMS = (
    (0, 768, (MASK_X, MASK_Y, MASK_Z1, MASK_Z2)),
    (768, 640, (MASK_Y, MASK_Z1, MASK_X, MASK_Z2)),
    (1408, 640, (MASK_Z1, MASK_X, MASK_Y, MASK_Z2)),
)

LENS = (1024, 512, 256, 128)
RECV_OFF = (0, 1024, 1536, 1792)
STAGE_OFF = (0, 0, 512, 768)


def _parity(v):
    return (v ^ (v >> 1) ^ (v >> 2) ^ (v >> 3)) & 1


def _functionals(masks):
    def par(x):
        return bin(x).count("1") & 1

    fs = []
    for j in range(4):
        for phi in range(16):
            if par(phi & masks[j]) == 1 and all(
                par(phi & masks[k]) == 0 for k in range(j + 1, 4)
            ):
                fs.append(phi)
                break
    assert len(fs) == j + 1
    return fs


def kernel(x, w_mat):
    m, k_shard = x.shape
    _, n = w_mat.shape

    def body(x_ref, w_ref, out_ref, pb_ref, acc_ref, stage_ref, recv_ref,
             rs_send_sems, rs_recv_sems, ag_send_sems, ag_recv_sems):
        d = lax.axis_index("i")

        barrier_sem = pltpu.get_barrier_semaphore()
        for mk in (MASK_X, MASK_Y, MASK_Z1, MASK_Z2):
            pl.semaphore_signal(
                barrier_sem, inc=1,
                device_id=(d ^ mk,), device_id_type=pl.DeviceIdType.MESH,
            )
        pl.semaphore_wait(barrier_sem, 4)

        sinfo = []
        for si, (c0, cw, masks) in enumerate(STREAMS):
            phis = _functionals(masks)
            h = [_parity(d & phi) for phi in phis]
            kstart = h[0] * 1024
            o = [0, h[1] * 512, h[1] * 512 + h[2] * 256]
            o.append(o[2] + h[3] * 128)
            sinfo.append(dict(c0=c0, cw=cw, masks=masks, h=h,
                              kstart=kstart, o=o))

        for j in range(4):
            for si, st in enumerate(sinfo):
                c0, cw, h = st["c0"], st["cw"], st["h"]
                partner = d ^ st["masks"][j]
                ln = LENS[j]
                if j == 0:
                    pb_ref[:, pl.ds(c0, cw)] = jnp.dot(
                        x_ref[...], w_ref[:, pl.ds(c0, cw)],
                        preferred_element_type=jnp.float32,
                    ).astype(jnp.bfloat16)
                    src = pb_ref.at[pl.ds((1 - h[0]) * 1024, 1024),
                                    pl.ds(c0, cw)]
                else:
                    prev = pltpu.make_async_remote_copy(
                        src_ref=pb_ref.at[pl.ds(0, LENS[j - 1]), pl.ds(c0, cw)],
                        dst_ref=recv_ref.at[pl.ds(RECV_OFF[j - 1], LENS[j - 1]),
                                            pl.ds(c0, cw)],
                        send_sem=rs_send_sems.at[si, j - 1],
                        recv_sem=rs_recv_sems.at[si, j - 1],
                        device_id=(d,),
                        device_id_type=pl.DeviceIdType.MESH,
                    )
                    prev.wait_recv()
                    rcv = recv_ref[pl.ds(RECV_OFF[j - 1], LENS[j - 1]),
                                   pl.ds(c0, cw)].astype(jnp.float32)
                    if j == 1:
                        base = pb_ref[pl.ds(st["kstart"] + 0, 1024),
                                      pl.ds(c0, cw)].astype(jnp.float32)
                        acc_ref[pl.ds(0, 1024), pl.ds(c0, cw)] = base + rcv
                    else:
                        cur = acc_ref[pl.ds(st["o"][j - 1], LENS[j - 1]),
                                      pl.ds(c0, cw)]
                        acc_ref[pl.ds(st["o"][j - 1], LENS[j - 1]),
                                pl.ds(c0, cw)] = cur + rcv
                    send_rel = st["o"][j - 1] + (1 - h[j]) * ln
                    stage_ref[pl.ds(STAGE_OFF[j], ln), pl.ds(c0, cw)] = (
                        acc_ref[pl.ds(send_rel, ln), pl.ds(c0, cw)]
                        .astype(jnp.bfloat16)
                    )
                    src = stage_ref.at[pl.ds(STAGE_OFF[j], ln), pl.ds(c0, cw)]
                rdma = pltpu.make_async_remote_copy(
                    src_ref=src,
                    dst_ref=recv_ref.at[pl.ds(RECV_OFF[j], ln), pl.ds(c0, cw)],
                    send_sem=rs_send_sems.at[si, j],
                    recv_sem=rs_recv_sems.at[si, j],
                    device_id=(partner,),
                    device_id_type=pl.DeviceIdType.MESH,
                )
                rdma.start()

        for si, st in enumerate(sinfo):
            c0, cw = st["c0"], st["cw"]
            last = pltpu.make_async_remote_copy(
                src_ref=pb_ref.at[pl.ds(0, 128), pl.ds(c0, cw)],
                dst_ref=recv_ref.at[pl.ds(RECV_OFF[3], 128), pl.ds(c0, cw)],
                send_sem=rs_send_sems.at[si, 3],
                recv_sem=rs_recv_sems.at[si, 3],
                device_id=(d,),
                device_id_type=pl.DeviceIdType.MESH,
            )
            last.wait_recv()
            fin = (acc_ref[pl.ds(st["o"][3], 128), pl.ds(c0, cw)]
                   + recv_ref[pl.ds(RECV_OFF[3], 128),
                              pl.ds(c0, cw)].astype(jnp.float32))
            g = st["kstart"] + st["o"][3]
            out_ref[pl.ds(g, 128), pl.ds(c0, cw)] = (
                jnp.maximum(fin, 0.0).astype(jnp.bfloat16)
            )

        for idx, j in enumerate((3, 2, 1, 0)):
            for si, st in enumerate(sinfo):
                c0, cw = st["c0"], st["cw"]
                partner = d ^ st["masks"][j]
                ln = LENS[j]
                blk_start = st["kstart"] + st["o"][j]
                if idx > 0:
                    pj = j + 1
                    prev = pltpu.make_async_remote_copy(
                        src_ref=out_ref.at[pl.ds(0, LENS[pj]), pl.ds(c0, cw)],
                        dst_ref=out_ref.at[
                            pl.ds(st["kstart"] + st["o"][pj], LENS[pj]),
                            pl.ds(c0, cw)],
                        send_sem=ag_send_sems.at[si, pj],
                        recv_sem=ag_recv_sems.at[si, pj],
                        device_id=(d,),
                        device_id_type=pl.DeviceIdType.MESH,
                    )
                    prev.wait_recv()
                rdma = pltpu.make_async_remote_copy(
                    src_ref=out_ref.at[pl.ds(blk_start, ln), pl.ds(c0, cw)],
                    dst_ref=out_ref.at[pl.ds(blk_start, ln), pl.ds(c0, cw)],
                    send_sem=ag_send_sems.at[si, j],
                    recv_sem=ag_recv_sems.at[si, j],
                    device_id=(partner,),
                    device_id_type=pl.DeviceIdType.MESH,
                )
                rdma.start()

        for si, st in enumerate(sinfo):
            c0, cw = st["c0"], st["cw"]
            fin = pltpu.make_async_remote_copy(
                src_ref=out_ref.at[pl.ds(0, 1024), pl.ds(c0, cw)],
                dst_ref=out_ref.at[pl.ds(st["kstart"], 1024), pl.ds(c0, cw)],
                send_sem=ag_send_sems.at[si, 0],
                recv_sem=ag_recv_sems.at[si, 0],
                device_id=(d,),
                device_id_type=pl.DeviceIdType.MESH,
            )
            fin.wait_recv()
        for si, st in enumerate(sinfo):
            c0, cw = st["c0"], st["cw"]
            for j in range(4):
                ln = LENS[j]
                rs_snd = pltpu.make_async_remote_copy(
                    src_ref=pb_ref.at[pl.ds(0, ln), pl.ds(c0, cw)],
                    dst_ref=recv_ref.at[pl.ds(RECV_OFF[j], ln), pl.ds(c0, cw)],
                    send_sem=rs_send_sems.at[si, j],
                    recv_sem=rs_recv_sems.at[si, j],
                    device_id=(d,),
                    device_id_type=pl.DeviceIdType.MESH,
                )
                rs_snd.wait_send()
                ag_snd = pltpu.make_async_remote_copy(
                    src_ref=out_ref.at[pl.ds(0, ln), pl.ds(c0, cw)],
                    dst_ref=out_ref.at[pl.ds(0, ln), pl.ds(c0, cw)],
                    send_sem=ag_send_sems.at[si, j],
                    recv_sem=ag_recv_sems.at[si, j],
                    device_id=(d,),
                    device_id_type=pl.DeviceIdType.MESH,
                )
                ag_snd.wait_send()

    n_streams = len(STREAMS)
    return pl.pallas_call(
        body,
        out_shape=jax.ShapeDtypeStruct((m, n), jnp.bfloat16),
        in_specs=[
            pl.BlockSpec(memory_space=pltpu.VMEM),
            pl.BlockSpec(memory_space=pltpu.VMEM),
        ],
        out_specs=pl.BlockSpec(memory_space=pltpu.VMEM),
        scratch_shapes=[
            pltpu.VMEM((m, n), jnp.bfloat16),
            pltpu.VMEM((1024, n), jnp.float32),
            pltpu.VMEM((1024, n), jnp.bfloat16),
            pltpu.VMEM((1920, n), jnp.bfloat16),
            pltpu.SemaphoreType.DMA((n_streams, 4)),
            pltpu.SemaphoreType.DMA((n_streams, 4)),
            pltpu.SemaphoreType.DMA((n_streams, 4)),
            pltpu.SemaphoreType.DMA((n_streams, 4)),
        ],
        compiler_params=pltpu.CompilerParams(collective_id=0),
    )(x, w_mat)


# device time: 95495 ns/iter; 2.4764x vs baseline; 1.0847x over previous
import jax
import jax.numpy as jnp
from jax import lax
from jax.experimental import pallas as pl
from jax.experimental.pallas import tpu as pltpu

N_DEV = 16
M = 2048

MASK_X, MASK_Y, MASK_Z1, MASK_Z2 = 1, 3, 4, 8

STREAMS = (
    (0, 768, (MASK_X, MASK_Y, MASK_Z1, MASK_Z2)),
    (768, 640, (MASK_Y, MASK_Z1, MASK_X, MASK_Z2)),
    (1408, 640, (MASK_Z1, MASK_X, MASK_Y, MASK_Z2)),
)

LENS = (1024, 512, 256, 128)
RECV_OFF = (0, 1024, 1536, 1792)
STAGE_OFF = (0, 0, 512, 768)


def _parity(v):
    return (v ^ (v >> 1) ^ (v >> 2) ^ (v >> 3)) & 1


def _functionals(masks):
    def par(x):
        return bin(x).count("1") & 1

    fs = []
    for j in range(4):
        for phi in range(16):
            if par(phi & masks[j]) == 1 and all(
                par(phi & masks[k]) == 0 for k in range(j + 1, 4)
            ):
                fs.append(phi)
                break
    assert len(fs) == 4
    return fs


def kernel(x, w_mat):
    m, k_shard = x.shape
    _, n = w_mat.shape

    def body(x_ref, w_ref, out_ref, pb_ref, acc_ref, stage_ref, recv_ref,
             rs_send_sems, rs_recv_sems, ag_send_sems, ag_recv_sems,
             rs_sub_send, rs_sub_recv, ag_sub_send, ag_sub_recv):
        d = lax.axis_index("i")

        barrier_sem = pltpu.get_barrier_semaphore()
        for mk in (MASK_X, MASK_Y, MASK_Z1, MASK_Z2):
            pl.semaphore_signal(
                barrier_sem, inc=1,
                device_id=(d ^ mk,), device_id_type=pl.DeviceIdType.MESH,
            )
        pl.semaphore_wait(barrier_sem, 4)

        sinfo = []
        for si, (c0, cw, masks) in enumerate(STREAMS):
            phis = _functionals(masks)
            h = [_parity(d & phi) for phi in phis]
            kstart = h[0] * 1024
            o = [0, h[1] * 512, h[1] * 512 + h[2] * 256]
            o.append(o[2] + h[3] * 128)
            f1m0 = bin(phis[1] & masks[0]).count("1") & 1
            h_r1 = h[1] ^ f1m0
            sinfo.append(dict(c0=c0, cw=cw, masks=masks, h=h,
                              kstart=kstart, o=o, h_r1=h_r1))

        def cols(st):
            return pl.ds(st["c0"], st["cw"])

        def rs_wait_recv(si, st, j, row_off, ln):
            desc = pltpu.make_async_remote_copy(
                src_ref=pb_ref.at[pl.ds(0, ln), cols(st)],
                dst_ref=recv_ref.at[pl.ds(row_off, ln), cols(st)],
                send_sem=rs_send_sems.at[si, j],
                recv_sem=rs_recv_sems.at[si, j],
                device_id=(d,),
                device_id_type=pl.DeviceIdType.MESH,
            )
            desc.wait_recv()

        for si, st in enumerate(sinfo):
            c0, cw, h = st["c0"], st["cw"], st["h"]
            partner = d ^ st["masks"][0]
            pb_ref[:, cols(st)] = jnp.dot(
                x_ref[...], w_ref[:, cols(st)],
                preferred_element_type=jnp.float32,
            ).astype(jnp.bfloat16)
            base = (1 - h[0]) * 1024
            a_off = (1 - st["h_r1"]) * 512
            b_off = st["h_r1"] * 512
            sub1 = pltpu.make_async_remote_copy(
                src_ref=pb_ref.at[pl.ds(base + a_off, 512), cols(st)],
                dst_ref=recv_ref.at[pl.ds(RECV_OFF[0] + a_off, 512), cols(st)],
                send_sem=rs_send_sems.at[si, 0],
                recv_sem=rs_recv_sems.at[si, 0],
                device_id=(partner,),
                device_id_type=pl.DeviceIdType.MESH,
            )
            sub1.start()
            sub2 = pltpu.make_async_remote_copy(
                src_ref=pb_ref.at[pl.ds(base + b_off, 512), cols(st)],
                dst_ref=recv_ref.at[pl.ds(RECV_OFF[0] + b_off, 512), cols(st)],
                send_sem=rs_sub_send.at[si],
                recv_sem=rs_sub_recv.at[si],
                device_id=(partner,),
                device_id_type=pl.DeviceIdType.MESH,
            )
            sub2.start()

        for si, st in enumerate(sinfo):
            c0, cw, h = st["c0"], st["cw"], st["h"]
            partner = d ^ st["masks"][1]
            a = (1 - h[1]) * 512
            rs_wait_recv(si, st, 0, RECV_OFF[0] + a, 512)
            acc_ref[pl.ds(a, 512), cols(st)] = (
                pb_ref[pl.ds(st["kstart"] + a, 512), cols(st)]
                .astype(jnp.float32)
                + recv_ref[pl.ds(RECV_OFF[0] + a, 512), cols(st)]
                .astype(jnp.float32)
            )
            stage_ref[pl.ds(STAGE_OFF[1], 512), cols(st)] = (
                acc_ref[pl.ds(a, 512), cols(st)].astype(jnp.bfloat16)
            )
            rdma = pltpu.make_async_remote_copy(
                src_ref=stage_ref.at[pl.ds(STAGE_OFF[1], 512), cols(st)],
                dst_ref=recv_ref.at[pl.ds(RECV_OFF[1], 512), cols(st)],
                send_sem=rs_send_sems.at[si, 1],
                recv_sem=rs_recv_sems.at[si, 1],
                device_id=(partner,),
                device_id_type=pl.DeviceIdType.MESH,
            )
            rdma.start()

        for si, st in enumerate(sinfo):
            c0, cw, h, o = st["c0"], st["cw"], st["h"], st["o"]
            partner = d ^ st["masks"][2]
            b = h[1] * 512
            sub2w = pltpu.make_async_remote_copy(
                src_ref=pb_ref.at[pl.ds(0, 512), cols(st)],
                dst_ref=recv_ref.at[pl.ds(RECV_OFF[0] + b, 512), cols(st)],
                send_sem=rs_sub_send.at[si],
                recv_sem=rs_sub_recv.at[si],
                device_id=(d,),
                device_id_type=pl.DeviceIdType.MESH,
            )
            sub2w.wait_recv()
            rs_wait_recv(si, st, 1, RECV_OFF[1], 512)
            acc_ref[pl.ds(o[1], 512), cols(st)] = (
                pb_ref[pl.ds(st["kstart"] + b, 512), cols(st)]
                .astype(jnp.float32)
                + recv_ref[pl.ds(RECV_OFF[0] + b, 512), cols(st)]
                .astype(jnp.float32)
                + recv_ref[pl.ds(RECV_OFF[1], 512), cols(st)]
                .astype(jnp.float32)
            )
            send_rel = o[1] + (1 - h[2]) * 256
            stage_ref[pl.ds(STAGE_OFF[2], 256), cols(st)] = (
                acc_ref[pl.ds(send_rel, 256), cols(st)].astype(jnp.bfloat16)
            )
            rdma = pltpu.make_async_remote_copy(
                src_ref=stage_ref.at[pl.ds(STAGE_OFF[2], 256), cols(st)],
                dst_ref=recv_ref.at[pl.ds(RECV_OFF[2], 256), cols(st)],
                send_sem=rs_send_sems.at[si, 2],
                recv_sem=rs_recv_sems.at[si, 2],
                device_id=(partner,),
                device_id_type=pl.DeviceIdType.MESH,
            )
            rdma.start()

        for si, st in enumerate(sinfo):
            c0, cw, h, o = st["c0"], st["cw"], st["h"], st["o"]
            partner = d ^ st["masks"][3]
            rs_wait_recv(si, st, 2, RECV_OFF[2], 256)
            acc_ref[pl.ds(o[2], 256), cols(st)] = (
                acc_ref[pl.ds(o[2], 256), cols(st)]
                + recv_ref[pl.ds(RECV_OFF[2], 256), cols(st)]
                .astype(jnp.float32)
            )
            send_rel = o[2] + (1 - h[3]) * 128
            stage_ref[pl.ds(STAGE_OFF[3], 128), cols(st)] = (
                acc_ref[pl.ds(send_rel, 128), cols(st)].astype(jnp.bfloat16)
            )
            rdma = pltpu.make_async_remote_copy(
                src_ref=stage_ref.at[pl.ds(STAGE_OFF[3], 128), cols(st)],
                dst_ref=recv_ref.at[pl.ds(RECV_OFF[3], 128), cols(st)],
                send_sem=rs_send_sems.at[si, 3],
                recv_sem=rs_recv_sems.at[si, 3],
                device_id=(partner,),
                device_id_type=pl.DeviceIdType.MESH,
            )
            rdma.start()

        for si, st in enumerate(sinfo):
            c0, cw, o = st["c0"], st["cw"], st["o"]
            rs_wait_recv(si, st, 3, RECV_OFF[3], 128)
            fin = (acc_ref[pl.ds(o[3], 128), cols(st)]
                   + recv_ref[pl.ds(RECV_OFF[3], 128), cols(st)]
                   .astype(jnp.float32))
            g = st["kstart"] + o[3]
            out_ref[pl.ds(g, 128), cols(st)] = (
                jnp.maximum(fin, 0.0).astype(jnp.bfloat16)
            )
            rdma = pltpu.make_async_remote_copy(
                src_ref=out_ref.at[pl.ds(g, 128), cols(st)],
                dst_ref=out_ref.at[pl.ds(g, 128), cols(st)],
                send_sem=ag_send_sems.at[si, 3],
                recv_sem=ag_recv_sems.at[si, 3],
                device_id=(d ^ st["masks"][3],),
                device_id_type=pl.DeviceIdType.MESH,
            )
            rdma.start()

        def ag_wait_recv(si, st, j, row_start, ln):
            desc = pltpu.make_async_remote_copy(
                src_ref=out_ref.at[pl.ds(0, ln), cols(st)],
                dst_ref=out_ref.at[pl.ds(row_start, ln), cols(st)],
                send_sem=ag_send_sems.at[si, j],
                recv_sem=ag_recv_sems.at[si, j],
                device_id=(d,),
                device_id_type=pl.DeviceIdType.MESH,
            )
            desc.wait_recv()

        for si, st in enumerate(sinfo):
            c0, cw, o = st["c0"], st["cw"], st["o"]
            ag_wait_recv(si, st, 3, st["kstart"] + o[3], 128)
            blk = st["kstart"] + o[2]
            rdma = pltpu.make_async_remote_copy(
                src_ref=out_ref.at[pl.ds(blk, 256), cols(st)],
                dst_ref=out_ref.at[pl.ds(blk, 256), cols(st)],
                send_sem=ag_send_sems.at[si, 2],
                recv_sem=ag_recv_sems.at[si, 2],
                device_id=(d ^ st["masks"][2],),
                device_id_type=pl.DeviceIdType.MESH,
            )
            rdma.start()

        for si, st in enumerate(sinfo):
            c0, cw, o = st["c0"], st["cw"], st["o"]
            ag_wait_recv(si, st, 2, st["kstart"] + o[2], 256)
            blk = st["kstart"] + o[1]
            rdma = pltpu.make_async_remote_copy(
                src_ref=out_ref.at[pl.ds(blk, 512), cols(st)],
                dst_ref=out_ref.at[pl.ds(blk, 512), cols(st)],
                send_sem=ag_send_sems.at[si, 1],
                recv_sem=ag_recv_sems.at[si, 1],
                device_id=(d ^ st["masks"][1],),
                device_id_type=pl.DeviceIdType.MESH,
            )
            rdma.start()
            sub_a = pltpu.make_async_remote_copy(
                src_ref=out_ref.at[pl.ds(blk, 512), cols(st)],
                dst_ref=out_ref.at[pl.ds(blk, 512), cols(st)],
                send_sem=ag_send_sems.at[si, 0],
                recv_sem=ag_recv_sems.at[si, 0],
                device_id=(d ^ st["masks"][0],),
                device_id_type=pl.DeviceIdType.MESH,
            )
            sub_a.start()

        for si, st in enumerate(sinfo):
            c0, cw, h, o = st["c0"], st["cw"], st["h"], st["o"]
            ag_wait_recv(si, st, 1, st["kstart"] + o[1], 512)
            rcvd = st["kstart"] + (1 - h[1]) * 512
            sub_b = pltpu.make_async_remote_copy(
                src_ref=out_ref.at[pl.ds(rcvd, 512), cols(st)],
                dst_ref=out_ref.at[pl.ds(rcvd, 512), cols(st)],
                send_sem=ag_sub_send.at[si],
                recv_sem=ag_sub_recv.at[si],
                device_id=(d ^ st["masks"][0],),
                device_id_type=pl.DeviceIdType.MESH,
            )
            sub_b.start()

        for si, st in enumerate(sinfo):
            ag_wait_recv(si, st, 0, st["kstart"], 512)
            fin_b = pltpu.make_async_remote_copy(
                src_ref=out_ref.at[pl.ds(0, 512), cols(st)],
                dst_ref=out_ref.at[pl.ds(st["kstart"], 512), cols(st)],
                send_sem=ag_sub_send.at[si],
                recv_sem=ag_sub_recv.at[si],
                device_id=(d,),
                device_id_type=pl.DeviceIdType.MESH,
            )
            fin_b.wait_recv()

        for si, st in enumerate(sinfo):
            for j, ln in ((0, 512), (1, 512), (2, 256), (3, 128)):
                snd = pltpu.make_async_remote_copy(
                    src_ref=pb_ref.at[pl.ds(0, ln), cols(st)],
                    dst_ref=recv_ref.at[pl.ds(RECV_OFF[j], ln), cols(st)],
                    send_sem=rs_send_sems.at[si, j],
                    recv_sem=rs_recv_sems.at[si, j],
                    device_id=(d,),
                    device_id_type=pl.DeviceIdType.MESH,
                )
                snd.wait_send()
                ag_snd = pltpu.make_async_remote_copy(
                    src_ref=out_ref.at[pl.ds(0, ln), cols(st)],
                    dst_ref=out_ref.at[pl.ds(0, ln), cols(st)],
                    send_sem=ag_send_sems.at[si, j],
                    recv_sem=ag_recv_sems.at[si, j],
                    device_id=(d,),
                    device_id_type=pl.DeviceIdType.MESH,
                )
                ag_snd.wait_send()
            for sub_sems in (rs_sub_send, ag_sub_send):
                sub = pltpu.make_async_remote_copy(
                    src_ref=pb_ref.at[pl.ds(0, 512), cols(st)],
                    dst_ref=recv_ref.at[pl.ds(0, 512), cols(st)],
                    send_sem=sub_sems.at[si],
                    recv_sem=rs_sub_recv.at[si],
                    device_id=(d,),
                    device_id_type=pl.DeviceIdType.MESH,
                )
                sub.wait_send()

    n_streams = len(STREAMS)
    return pl.pallas_call(
        body,
        out_shape=jax.ShapeDtypeStruct((m, n), jnp.bfloat16),
        in_specs=[
            pl.BlockSpec(memory_space=pltpu.VMEM),
            pl.BlockSpec(memory_space=pltpu.VMEM),
        ],
        out_specs=pl.BlockSpec(memory_space=pltpu.VMEM),
        scratch_shapes=[
            pltpu.VMEM((m, n), jnp.bfloat16),
            pltpu.VMEM((1024, n), jnp.float32),
            pltpu.VMEM((1024, n), jnp.bfloat16),
            pltpu.VMEM((1920, n), jnp.bfloat16),
            pltpu.SemaphoreType.DMA((n_streams, 4)),
            pltpu.SemaphoreType.DMA((n_streams, 4)),
            pltpu.SemaphoreType.DMA((n_streams, 4)),
            pltpu.SemaphoreType.DMA((n_streams, 4)),
            pltpu.SemaphoreType.DMA((n_streams,)),
            pltpu.SemaphoreType.DMA((n_streams,)),
            pltpu.SemaphoreType.DMA((n_streams,)),
            pltpu.SemaphoreType.DMA((n_streams,)),
        ],
        compiler_params=pltpu.CompilerParams(collective_id=0),
    )(x, w_mat)


# device time: 95280 ns/iter; 2.4820x vs baseline; 1.0023x over previous
import jax
import jax.numpy as jnp
from jax import lax
from jax.experimental import pallas as pl
from jax.experimental.pallas import tpu as pltpu

N_DEV = 16
M = 2048

MASK_X, MASK_Y, MASK_Z1, MASK_Z2 = 1, 3, 4, 8

STREAMS = (
    (0, 768, (MASK_X, MASK_Y, MASK_Z1, MASK_Z2)),
    (768, 640, (MASK_Y, MASK_Z1, MASK_X, MASK_Z2)),
    (1408, 640, (MASK_Z1, MASK_X, MASK_Y, MASK_Z2)),
)

LENS = (1024, 512, 256, 128)
RECV_OFF = (0, 1024, 1536, 1792)
STAGE_OFF = (0, 0, 512, 768)


def _parity(v):
    return (v ^ (v >> 1) ^ (v >> 2) ^ (v >> 3)) & 1


def _functionals(masks):
    def par(x):
        return bin(x).count("1") & 1

    fs = []
    for j in range(4):
        for phi in range(16):
            if par(phi & masks[j]) == 1 and all(
                par(phi & masks[k]) == 0 for k in range(j + 1, 4)
            ):
                fs.append(phi)
                break
    assert len(fs) == 4
    return fs


def kernel(x, w_mat):
    m, k_shard = x.shape
    _, n = w_mat.shape

    def body(x_ref, w_ref, out_ref, pb_ref, acc_ref, stage_ref, recv_ref,
             rs_send_sems, rs_recv_sems, ag_send_sems, ag_recv_sems,
             rs_sub_send, rs_sub_recv, ag_sub_send, ag_sub_recv):
        d = lax.axis_index("i")

        barrier_sem = pltpu.get_barrier_semaphore()
        for mk in (MASK_X, MASK_Y, MASK_Z1, MASK_Z2):
            pl.semaphore_signal(
                barrier_sem, inc=1,
                device_id=(d ^ mk,), device_id_type=pl.DeviceIdType.MESH,
            )
        pl.semaphore_wait(barrier_sem, 4)

        sinfo = []
        for si, (c0, cw, masks) in enumerate(STREAMS):
            phis = _functionals(masks)
            h = [_parity(d & phi) for phi in phis]
            kstart = h[0] * 1024
            o = [0, h[1] * 512, h[1] * 512 + h[2] * 256]
            o.append(o[2] + h[3] * 128)
            f1m0 = bin(phis[1] & masks[0]).count("1") & 1
            h_r1 = h[1] ^ f1m0
            sinfo.append(dict(c0=c0, cw=cw, masks=masks, h=h,
                              kstart=kstart, o=o, h_r1=h_r1))

        def cols(st):
            return pl.ds(st["c0"], st["cw"])

        def rs_wait_recv(si, st, j, row_off, ln):
            desc = pltpu.make_async_remote_copy(
                src_ref=pb_ref.at[pl.ds(0, ln), cols(st)],
                dst_ref=recv_ref.at[pl.ds(row_off, ln), cols(st)],
                send_sem=rs_send_sems.at[si, j],
                recv_sem=rs_recv_sems.at[si, j],
                device_id=(d,),
                device_id_type=pl.DeviceIdType.MESH,
            )
            desc.wait_recv()

        for si, st in enumerate(sinfo):
            c0, cw, h = st["c0"], st["cw"], st["h"]
            partner = d ^ st["masks"][0]
            pb_ref[:, cols(st)] = jnp.dot(
                x_ref[...], w_ref[:, cols(st)],
                preferred_element_type=jnp.float32,
            ).astype(jnp.bfloat16)
            base = (1 - h[0]) * 1024
            a_off = (1 - st["h_r1"]) * 512
            b_off = st["h_r1"] * 512
            sub1 = pltpu.make_async_remote_copy(
                src_ref=pb_ref.at[pl.ds(base + a_off, 512), cols(st)],
                dst_ref=recv_ref.at[pl.ds(RECV_OFF[0] + a_off, 512), cols(st)],
                send_sem=rs_send_sems.at[si, 0],
                recv_sem=rs_recv_sems.at[si, 0],
                device_id=(partner,),
                device_id_type=pl.DeviceIdType.MESH,
            )
            sub1.start()
            sub2 = pltpu.make_async_remote_copy(
                src_ref=pb_ref.at[pl.ds(base + b_off, 512), cols(st)],
                dst_ref=recv_ref.at[pl.ds(RECV_OFF[0] + b_off, 512), cols(st)],
                send_sem=rs_sub_send.at[si],
                recv_sem=rs_sub_recv.at[si],
                device_id=(partner,),
                device_id_type=pl.DeviceIdType.MESH,
            )
            sub2.start()

        for si, st in enumerate(sinfo):
            c0, cw, h = st["c0"], st["cw"], st["h"]
            partner = d ^ st["masks"][1]
            a = (1 - h[1]) * 512
            rs_wait_recv(si, st, 0, RECV_OFF[0] + a, 512)
            acc_ref[pl.ds(a, 512), cols(st)] = (
                pb_ref[pl.ds(st["kstart"] + a, 512), cols(st)]
                + recv_ref[pl.ds(RECV_OFF[0] + a, 512), cols(st)]
            )
            rdma = pltpu.make_async_remote_copy(
                src_ref=acc_ref.at[pl.ds(a, 512), cols(st)],
                dst_ref=recv_ref.at[pl.ds(RECV_OFF[1], 512), cols(st)],
                send_sem=rs_send_sems.at[si, 1],
                recv_sem=rs_recv_sems.at[si, 1],
                device_id=(partner,),
                device_id_type=pl.DeviceIdType.MESH,
            )
            rdma.start()

        for si, st in enumerate(sinfo):
            c0, cw, h, o = st["c0"], st["cw"], st["h"], st["o"]
            partner = d ^ st["masks"][2]
            b = h[1] * 512
            sub2w = pltpu.make_async_remote_copy(
                src_ref=pb_ref.at[pl.ds(0, 512), cols(st)],
                dst_ref=recv_ref.at[pl.ds(RECV_OFF[0] + b, 512), cols(st)],
                send_sem=rs_sub_send.at[si],
                recv_sem=rs_sub_recv.at[si],
                device_id=(d,),
                device_id_type=pl.DeviceIdType.MESH,
            )
            sub2w.wait_recv()
            rs_wait_recv(si, st, 1, RECV_OFF[1], 512)
            acc_ref[pl.ds(o[1], 512), cols(st)] = (
                pb_ref[pl.ds(st["kstart"] + b, 512), cols(st)]
                + recv_ref[pl.ds(RECV_OFF[0] + b, 512), cols(st)]
                + recv_ref[pl.ds(RECV_OFF[1], 512), cols(st)]
            )
            send_rel = o[1] + (1 - h[2]) * 256
            rdma = pltpu.make_async_remote_copy(
                src_ref=acc_ref.at[pl.ds(send_rel, 256), cols(st)],
                dst_ref=recv_ref.at[pl.ds(RECV_OFF[2], 256), cols(st)],
                send_sem=rs_send_sems.at[si, 2],
                recv_sem=rs_recv_sems.at[si, 2],
                device_id=(partner,),
                device_id_type=pl.DeviceIdType.MESH,
            )
            rdma.start()

        for si, st in enumerate(sinfo):
            c0, cw, h, o = st["c0"], st["cw"], st["h"], st["o"]
            partner = d ^ st["masks"][3]
            rs_wait_recv(si, st, 2, RECV_OFF[2], 256)
            acc_ref[pl.ds(o[2], 256), cols(st)] = (
                acc_ref[pl.ds(o[2], 256), cols(st)]
                + recv_ref[pl.ds(RECV_OFF[2], 256), cols(st)]
            )
            send_rel = o[2] + (1 - h[3]) * 128
            rdma = pltpu.make_async_remote_copy(
                src_ref=acc_ref.at[pl.ds(send_rel, 128), cols(st)],
                dst_ref=recv_ref.at[pl.ds(RECV_OFF[3], 128), cols(st)],
                send_sem=rs_send_sems.at[si, 3],
                recv_sem=rs_recv_sems.at[si, 3],
                device_id=(partner,),
                device_id_type=pl.DeviceIdType.MESH,
            )
            rdma.start()

        for si, st in enumerate(sinfo):
            c0, cw, o = st["c0"], st["cw"], st["o"]
            rs_wait_recv(si, st, 3, RECV_OFF[3], 128)
            fin = (acc_ref[pl.ds(o[3], 128), cols(st)]
                   + recv_ref[pl.ds(RECV_OFF[3], 128), cols(st)])
            g = st["kstart"] + o[3]
            out_ref[pl.ds(g, 128), cols(st)] = jnp.maximum(
                fin, jnp.bfloat16(0.0)
            )
            rdma = pltpu.make_async_remote_copy(
                src_ref=out_ref.at[pl.ds(g, 128), cols(st)],
                dst_ref=out_ref.at[pl.ds(g, 128), cols(st)],
                send_sem=ag_send_sems.at[si, 3],
                recv_sem=ag_recv_sems.at[si, 3],
                device_id=(d ^ st["masks"][3],),
                device_id_type=pl.DeviceIdType.MESH,
            )
            rdma.start()

        def ag_wait_recv(si, st, j, row_start, ln):
            desc = pltpu.make_async_remote_copy(
                src_ref=out_ref.at[pl.ds(0, ln), cols(st)],
                dst_ref=out_ref.at[pl.ds(row_start, ln), cols(st)],
                send_sem=ag_send_sems.at[si, j],
                recv_sem=ag_recv_sems.at[si, j],
                device_id=(d,),
                device_id_type=pl.DeviceIdType.MESH,
            )
            desc.wait_recv()

        for si, st in enumerate(sinfo):
            c0, cw, o = st["c0"], st["cw"], st["o"]
            ag_wait_recv(si, st, 3, st["kstart"] + o[3], 128)
            blk = st["kstart"] + o[2]
            rdma = pltpu.make_async_remote_copy(
                src_ref=out_ref.at[pl.ds(blk, 256), cols(st)],
                dst_ref=out_ref.at[pl.ds(blk, 256), cols(st)],
                send_sem=ag_send_sems.at[si, 2],
                recv_sem=ag_recv_sems.at[si, 2],
                device_id=(d ^ st["masks"][2],),
                device_id_type=pl.DeviceIdType.MESH,
            )
            rdma.start()

        for si, st in enumerate(sinfo):
            c0, cw, o = st["c0"], st["cw"], st["o"]
            ag_wait_recv(si, st, 2, st["kstart"] + o[2], 256)
            blk = st["kstart"] + o[1]
            rdma = pltpu.make_async_remote_copy(
                src_ref=out_ref.at[pl.ds(blk, 512), cols(st)],
                dst_ref=out_ref.at[pl.ds(blk, 512), cols(st)],
                send_sem=ag_send_sems.at[si, 1],
                recv_sem=ag_recv_sems.at[si, 1],
                device_id=(d ^ st["masks"][1],),
                device_id_type=pl.DeviceIdType.MESH,
            )
            rdma.start()
            sub_a = pltpu.make_async_remote_copy(
                src_ref=out_ref.at[pl.ds(blk, 512), cols(st)],
                dst_ref=out_ref.at[pl.ds(blk, 512), cols(st)],
                send_sem=ag_send_sems.at[si, 0],
                recv_sem=ag_recv_sems.at[si, 0],
                device_id=(d ^ st["masks"][0],),
                device_id_type=pl.DeviceIdType.MESH,
            )
            sub_a.start()

        for si, st in enumerate(sinfo):
            c0, cw, h, o = st["c0"], st["cw"], st["h"], st["o"]
            ag_wait_recv(si, st, 1, st["kstart"] + o[1], 512)
            rcvd = st["kstart"] + (1 - h[1]) * 512
            sub_b = pltpu.make_async_remote_copy(
                src_ref=out_ref.at[pl.ds(rcvd, 512), cols(st)],
                dst_ref=out_ref.at[pl.ds(rcvd, 512), cols(st)],
                send_sem=ag_sub_send.at[si],
                recv_sem=ag_sub_recv.at[si],
                device_id=(d ^ st["masks"][0],),
                device_id_type=pl.DeviceIdType.MESH,
            )
            sub_b.start()

        for si, st in enumerate(sinfo):
            ag_wait_recv(si, st, 0, st["kstart"], 512)
            fin_b = pltpu.make_async_remote_copy(
                src_ref=out_ref.at[pl.ds(0, 512), cols(st)],
                dst_ref=out_ref.at[pl.ds(st["kstart"], 512), cols(st)],
                send_sem=ag_sub_send.at[si],
                recv_sem=ag_sub_recv.at[si],
                device_id=(d,),
                device_id_type=pl.DeviceIdType.MESH,
            )
            fin_b.wait_recv()

        for si, st in enumerate(sinfo):
            for j, ln in ((0, 512), (1, 512), (2, 256), (3, 128)):
                snd = pltpu.make_async_remote_copy(
                    src_ref=pb_ref.at[pl.ds(0, ln), cols(st)],
                    dst_ref=recv_ref.at[pl.ds(RECV_OFF[j], ln), cols(st)],
                    send_sem=rs_send_sems.at[si, j],
                    recv_sem=rs_recv_sems.at[si, j],
                    device_id=(d,),
                    device_id_type=pl.DeviceIdType.MESH,
                )
                snd.wait_send()
                ag_snd = pltpu.make_async_remote_copy(
                    src_ref=out_ref.at[pl.ds(0, ln), cols(st)],
                    dst_ref=out_ref.at[pl.ds(0, ln), cols(st)],
                    send_sem=ag_send_sems.at[si, j],
                    recv_sem=ag_recv_sems.at[si, j],
                    device_id=(d,),
                    device_id_type=pl.DeviceIdType.MESH,
                )
                ag_snd.wait_send()
            for sub_sems in (rs_sub_send, ag_sub_send):
                sub = pltpu.make_async_remote_copy(
                    src_ref=pb_ref.at[pl.ds(0, 512), cols(st)],
                    dst_ref=recv_ref.at[pl.ds(0, 512), cols(st)],
                    send_sem=sub_sems.at[si],
                    recv_sem=rs_sub_recv.at[si],
                    device_id=(d,),
                    device_id_type=pl.DeviceIdType.MESH,
                )
                sub.wait_send()

    n_streams = len(STREAMS)
    return pl.pallas_call(
        body,
        out_shape=jax.ShapeDtypeStruct((m, n), jnp.bfloat16),
        in_specs=[
            pl.BlockSpec(memory_space=pltpu.VMEM),
            pl.BlockSpec(memory_space=pltpu.VMEM),
        ],
        out_specs=pl.BlockSpec(memory_space=pltpu.VMEM),
        scratch_shapes=[
            pltpu.VMEM((m, n), jnp.bfloat16),
            pltpu.VMEM((1024, n), jnp.bfloat16),
            pltpu.VMEM((8, n), jnp.bfloat16),
            pltpu.VMEM((1920, n), jnp.bfloat16),
            pltpu.SemaphoreType.DMA((n_streams, 4)),
            pltpu.SemaphoreType.DMA((n_streams, 4)),
            pltpu.SemaphoreType.DMA((n_streams, 4)),
            pltpu.SemaphoreType.DMA((n_streams, 4)),
            pltpu.SemaphoreType.DMA((n_streams,)),
            pltpu.SemaphoreType.DMA((n_streams,)),
            pltpu.SemaphoreType.DMA((n_streams,)),
            pltpu.SemaphoreType.DMA((n_streams,)),
        ],
        compiler_params=pltpu.CompilerParams(collective_id=0),
    )(x, w_mat)


# device time: 95254 ns/iter; 2.4826x vs baseline; 1.0003x over previous
import jax
import jax.numpy as jnp
from jax import lax
from jax.experimental import pallas as pl
from jax.experimental.pallas import tpu as pltpu

N_DEV = 16
M = 2048

MASK_X, MASK_Y, MASK_Z1, MASK_Z2 = 1, 3, 4, 8

STREAMS = (
    (0, 768, (MASK_X, MASK_Y, MASK_Z1, MASK_Z2)),
    (768, 640, (MASK_Y, MASK_Z1, MASK_X, MASK_Z2)),
    (1408, 640, (MASK_Z1, MASK_X, MASK_Y, MASK_Z2)),
)

LENS = (1024, 512, 256, 128)
RECV_OFF = (0, 1024, 1536, 1792)


def _parity(v):
    return (v ^ (v >> 1) ^ (v >> 2) ^ (v >> 3)) & 1


def _functionals(masks):
    def par(x):
        return bin(x).count("1") & 1

    fs = []
    for j in range(4):
        for phi in range(16):
            if par(phi & masks[j]) == 1 and all(
                par(phi & masks[k]) == 0 for k in range(j + 1, 4)
            ):
                fs.append(phi)
                break
    assert len(fs) == 4
    return fs


def kernel(x, w_mat):
    m, k_shard = x.shape
    _, n = w_mat.shape

    def body(x_ref, w_ref, out_ref, pb_ref, acc_ref, recv_ref,
             rs_send_sems, rs_recv_sems, ag_send_sems, ag_recv_sems,
             rs_sub_send, rs_sub_recv, ag_sub_send, ag_sub_recv):
        d = lax.axis_index("i")

        barrier_sem = pltpu.get_barrier_semaphore()
        for mk in (MASK_X, MASK_Y, MASK_Z1, MASK_Z2):
            pl.semaphore_signal(
                barrier_sem, inc=1,
                device_id=(d ^ mk,), device_id_type=pl.DeviceIdType.MESH,
            )
        pl.semaphore_wait(barrier_sem, 4)

        sinfo = []
        for si, (c0, cw, masks) in enumerate(STREAMS):
            phis = _functionals(masks)
            h = [_parity(d & phi) for phi in phis]
            kstart = h[0] * 1024
            o = [0, h[1] * 512, h[1] * 512 + h[2] * 256]
            o.append(o[2] + h[3] * 128)
            f1m0 = bin(phis[1] & masks[0]).count("1") & 1
            h_r1 = h[1] ^ f1m0
            sinfo.append(dict(c0=c0, cw=cw, masks=masks, h=h,
                              kstart=kstart, o=o, h_r1=h_r1))

        def cols(st):
            return pl.ds(st["c0"], st["cw"])

        def rs_wait_recv(si, st, j, row_off, ln):
            desc = pltpu.make_async_remote_copy(
                src_ref=pb_ref.at[pl.ds(0, ln), cols(st)],
                dst_ref=recv_ref.at[pl.ds(row_off, ln), cols(st)],
                send_sem=rs_send_sems.at[si, j],
                recv_sem=rs_recv_sems.at[si, j],
                device_id=(d,),
                device_id_type=pl.DeviceIdType.MESH,
            )
            desc.wait_recv()

        for si, st in enumerate(sinfo):
            c0, cw, h = st["c0"], st["cw"], st["h"]
            partner = d ^ st["masks"][0]
            pb_ref[:, cols(st)] = jnp.dot(
                x_ref[...], w_ref[:, cols(st)],
                preferred_element_type=jnp.float32,
            ).astype(jnp.bfloat16)
            base = (1 - h[0]) * 1024
            a_off = (1 - st["h_r1"]) * 512
            b_off = st["h_r1"] * 512
            sub1 = pltpu.make_async_remote_copy(
                src_ref=pb_ref.at[pl.ds(base + a_off, 512), cols(st)],
                dst_ref=recv_ref.at[pl.ds(RECV_OFF[0] + a_off, 512), cols(st)],
                send_sem=rs_send_sems.at[si, 0],
                recv_sem=rs_recv_sems.at[si, 0],
                device_id=(partner,),
                device_id_type=pl.DeviceIdType.MESH,
            )
            sub1.start()
            sub2 = pltpu.make_async_remote_copy(
                src_ref=pb_ref.at[pl.ds(base + b_off, 512), cols(st)],
                dst_ref=recv_ref.at[pl.ds(RECV_OFF[0] + b_off, 512), cols(st)],
                send_sem=rs_sub_send.at[si],
                recv_sem=rs_sub_recv.at[si],
                device_id=(partner,),
                device_id_type=pl.DeviceIdType.MESH,
            )
            sub2.start()

        for si, st in enumerate(sinfo):
            c0, cw, h = st["c0"], st["cw"], st["h"]
            partner = d ^ st["masks"][1]
            a = (1 - h[1]) * 512
            rs_wait_recv(si, st, 0, RECV_OFF[0] + a, 512)
            acc_ref[pl.ds(a, 512), cols(st)] = (
                pb_ref[pl.ds(st["kstart"] + a, 512), cols(st)]
                + recv_ref[pl.ds(RECV_OFF[0] + a, 512), cols(st)]
            )
            rdma = pltpu.make_async_remote_copy(
                src_ref=acc_ref.at[pl.ds(a, 512), cols(st)],
                dst_ref=recv_ref.at[pl.ds(RECV_OFF[1], 512), cols(st)],
                send_sem=rs_send_sems.at[si, 1],
                recv_sem=rs_recv_sems.at[si, 1],
                device_id=(partner,),
                device_id_type=pl.DeviceIdType.MESH,
            )
            rdma.start()

        for si, st in enumerate(sinfo):
            c0, cw, h, o = st["c0"], st["cw"], st["h"], st["o"]
            partner = d ^ st["masks"][2]
            b = h[1] * 512
            sub2w = pltpu.make_async_remote_copy(
                src_ref=pb_ref.at[pl.ds(0, 512), cols(st)],
                dst_ref=recv_ref.at[pl.ds(RECV_OFF[0] + b, 512), cols(st)],
                send_sem=rs_sub_send.at[si],
                recv_sem=rs_sub_recv.at[si],
                device_id=(d,),
                device_id_type=pl.DeviceIdType.MESH,
            )
            sub2w.wait_recv()
            rs_wait_recv(si, st, 1, RECV_OFF[1], 512)
            acc_ref[pl.ds(o[1], 512), cols(st)] = (
                pb_ref[pl.ds(st["kstart"] + b, 512), cols(st)]
                + recv_ref[pl.ds(RECV_OFF[0] + b, 512), cols(st)]
                + recv_ref[pl.ds(RECV_OFF[1], 512), cols(st)]
            )
            send_rel = o[1] + (1 - h[2]) * 256
            rdma = pltpu.make_async_remote_copy(
                src_ref=acc_ref.at[pl.ds(send_rel, 256), cols(st)],
                dst_ref=recv_ref.at[pl.ds(RECV_OFF[2], 256), cols(st)],
                send_sem=rs_send_sems.at[si, 2],
                recv_sem=rs_recv_sems.at[si, 2],
                device_id=(partner,),
                device_id_type=pl.DeviceIdType.MESH,
            )
            rdma.start()

        for si, st in enumerate(sinfo):
            c0, cw, h, o = st["c0"], st["cw"], st["h"], st["o"]
            partner = d ^ st["masks"][3]
            rs_wait_recv(si, st, 2, RECV_OFF[2], 256)
            acc_ref[pl.ds(o[2], 256), cols(st)] = (
                acc_ref[pl.ds(o[2], 256), cols(st)]
                + recv_ref[pl.ds(RECV_OFF[2], 256), cols(st)]
            )
            send_rel = o[2] + (1 - h[3]) * 128
            rdma = pltpu.make_async_remote_copy(
                src_ref=acc_ref.at[pl.ds(send_rel, 128), cols(st)],
                dst_ref=recv_ref.at[pl.ds(RECV_OFF[3], 128), cols(st)],
                send_sem=rs_send_sems.at[si, 3],
                recv_sem=rs_recv_sems.at[si, 3],
                device_id=(partner,),
                device_id_type=pl.DeviceIdType.MESH,
            )
            rdma.start()

        for si, st in enumerate(sinfo):
            c0, cw, o = st["c0"], st["cw"], st["o"]
            rs_wait_recv(si, st, 3, RECV_OFF[3], 128)
            fin = (acc_ref[pl.ds(o[3], 128), cols(st)]
                   + recv_ref[pl.ds(RECV_OFF[3], 128), cols(st)])
            g = st["kstart"] + o[3]
            out_ref[pl.ds(g, 128), cols(st)] = jnp.maximum(
                fin, jnp.bfloat16(0.0)
            )
            rdma = pltpu.make_async_remote_copy(
                src_ref=out_ref.at[pl.ds(g, 128), cols(st)],
                dst_ref=out_ref.at[pl.ds(g, 128), cols(st)],
                send_sem=ag_send_sems.at[si, 3],
                recv_sem=ag_recv_sems.at[si, 3],
                device_id=(d ^ st["masks"][3],),
                device_id_type=pl.DeviceIdType.MESH,
            )
            rdma.start()

        def ag_wait_recv(si, st, j, row_start, ln):
            desc = pltpu.make_async_remote_copy(
                src_ref=out_ref.at[pl.ds(0, ln), cols(st)],
                dst_ref=out_ref.at[pl.ds(row_start, ln), cols(st)],
                send_sem=ag_send_sems.at[si, j],
                recv_sem=ag_recv_sems.at[si, j],
                device_id=(d,),
                device_id_type=pl.DeviceIdType.MESH,
            )
            desc.wait_recv()

        for si, st in enumerate(sinfo):
            c0, cw, o = st["c0"], st["cw"], st["o"]
            ag_wait_recv(si, st, 3, st["kstart"] + o[3], 128)
            blk = st["kstart"] + o[2]
            rdma = pltpu.make_async_remote_copy(
                src_ref=out_ref.at[pl.ds(blk, 256), cols(st)],
                dst_ref=out_ref.at[pl.ds(blk, 256), cols(st)],
                send_sem=ag_send_sems.at[si, 2],
                recv_sem=ag_recv_sems.at[si, 2],
                device_id=(d ^ st["masks"][2],),
                device_id_type=pl.DeviceIdType.MESH,
            )
            rdma.start()

        for si, st in enumerate(sinfo):
            c0, cw, o = st["c0"], st["cw"], st["o"]
            ag_wait_recv(si, st, 2, st["kstart"] + o[2], 256)
            blk = st["kstart"] + o[1]
            rdma = pltpu.make_async_remote_copy(
                src_ref=out_ref.at[pl.ds(blk, 512), cols(st)],
                dst_ref=out_ref.at[pl.ds(blk, 512), cols(st)],
                send_sem=ag_send_sems.at[si, 1],
                recv_sem=ag_recv_sems.at[si, 1],
                device_id=(d ^ st["masks"][1],),
                device_id_type=pl.DeviceIdType.MESH,
            )
            rdma.start()
            sub_a = pltpu.make_async_remote_copy(
                src_ref=out_ref.at[pl.ds(blk, 512), cols(st)],
                dst_ref=out_ref.at[pl.ds(blk, 512), cols(st)],
                send_sem=ag_send_sems.at[si, 0],
                recv_sem=ag_recv_sems.at[si, 0],
                device_id=(d ^ st["masks"][0],),
                device_id_type=pl.DeviceIdType.MESH,
            )
            sub_a.start()

        for si, st in enumerate(sinfo):
            c0, cw, h, o = st["c0"], st["cw"], st["h"], st["o"]
            ag_wait_recv(si, st, 1, st["kstart"] + o[1], 512)
            rcvd = st["kstart"] + (1 - h[1]) * 512
            sub_b = pltpu.make_async_remote_copy(
                src_ref=out_ref.at[pl.ds(rcvd, 512), cols(st)],
                dst_ref=out_ref.at[pl.ds(rcvd, 512), cols(st)],
                send_sem=ag_sub_send.at[si],
                recv_sem=ag_sub_recv.at[si],
                device_id=(d ^ st["masks"][0],),
                device_id_type=pl.DeviceIdType.MESH,
            )
            sub_b.start()

        for si, st in enumerate(sinfo):
            ag_wait_recv(si, st, 0, st["kstart"], 512)
            fin_b = pltpu.make_async_remote_copy(
                src_ref=out_ref.at[pl.ds(0, 512), cols(st)],
                dst_ref=out_ref.at[pl.ds(st["kstart"], 512), cols(st)],
                send_sem=ag_sub_send.at[si],
                recv_sem=ag_sub_recv.at[si],
                device_id=(d,),
                device_id_type=pl.DeviceIdType.MESH,
            )
            fin_b.wait_recv()

        for si, st in enumerate(sinfo):
            for j, ln in ((0, 512), (1, 512), (2, 256), (3, 128)):
                snd = pltpu.make_async_remote_copy(
                    src_ref=pb_ref.at[pl.ds(0, ln), cols(st)],
                    dst_ref=recv_ref.at[pl.ds(RECV_OFF[j], ln), cols(st)],
                    send_sem=rs_send_sems.at[si, j],
                    recv_sem=rs_recv_sems.at[si, j],
                    device_id=(d,),
                    device_id_type=pl.DeviceIdType.MESH,
                )
                snd.wait_send()
                ag_snd = pltpu.make_async_remote_copy(
                    src_ref=out_ref.at[pl.ds(0, ln), cols(st)],
                    dst_ref=out_ref.at[pl.ds(0, ln), cols(st)],
                    send_sem=ag_send_sems.at[si, j],
                    recv_sem=ag_recv_sems.at[si, j],
                    device_id=(d,),
                    device_id_type=pl.DeviceIdType.MESH,
                )
                ag_snd.wait_send()
            for sub_sems in (rs_sub_send, ag_sub_send):
                sub = pltpu.make_async_remote_copy(
                    src_ref=pb_ref.at[pl.ds(0, 512), cols(st)],
                    dst_ref=recv_ref.at[pl.ds(0, 512), cols(st)],
                    send_sem=sub_sems.at[si],
                    recv_sem=rs_sub_recv.at[si],
                    device_id=(d,),
                    device_id_type=pl.DeviceIdType.MESH,
                )
                sub.wait_send()

    n_streams = len(STREAMS)
    return pl.pallas_call(
        body,
        out_shape=jax.ShapeDtypeStruct((m, n), jnp.bfloat16),
        in_specs=[
            pl.BlockSpec(memory_space=pltpu.VMEM),
            pl.BlockSpec(memory_space=pltpu.VMEM),
        ],
        out_specs=pl.BlockSpec(memory_space=pltpu.VMEM),
        scratch_shapes=[
            pltpu.VMEM((m, n), jnp.bfloat16),
            pltpu.VMEM((1024, n), jnp.bfloat16),
            pltpu.VMEM((1920, n), jnp.bfloat16),
            pltpu.SemaphoreType.DMA((n_streams, 4)),
            pltpu.SemaphoreType.DMA((n_streams, 4)),
            pltpu.SemaphoreType.DMA((n_streams, 4)),
            pltpu.SemaphoreType.DMA((n_streams, 4)),
            pltpu.SemaphoreType.DMA((n_streams,)),
            pltpu.SemaphoreType.DMA((n_streams,)),
            pltpu.SemaphoreType.DMA((n_streams,)),
            pltpu.SemaphoreType.DMA((n_streams,)),
        ],
        compiler_params=pltpu.CompilerParams(collective_id=0),
    )(x, w_mat)
